# baseline (speedup 1.0000x reference)
import functools

import jax
import jax.numpy as jnp
from jax.experimental import pallas as pl
from jax.experimental.pallas import tpu as pltpu

N = 5000
NPAD = 5120
B = 512
NB = NPAD // B
THRESHOLD = 0.5
MAX_SIZE = 300


def _nms_step(bT_ref, out_ref, keep_ref):
    k = pl.program_id(0)

    @pl.when(k == 0)
    def _init():
        keep_ref[...] = jnp.zeros_like(keep_ref)

    base = k * B
    cx1 = jnp.reshape(bT_ref[0:1, pl.ds(base, B)], (B, 1))
    cy1 = jnp.reshape(bT_ref[1:2, pl.ds(base, B)], (B, 1))
    cx2 = jnp.reshape(bT_ref[2:3, pl.ds(base, B)], (B, 1))
    cy2 = jnp.reshape(bT_ref[3:4, pl.ds(base, B)], (B, 1))
    careas = (cx2 - cx1) * (cy2 - cy1)

    def pbody(jb, acc):
        jbase = jb * B
        jx1 = bT_ref[0:1, pl.ds(jbase, B)]
        jy1 = bT_ref[1:2, pl.ds(jbase, B)]
        jx2 = bT_ref[2:3, pl.ds(jbase, B)]
        jy2 = bT_ref[3:4, pl.ds(jbase, B)]
        jareas = (jx2 - jx1) * (jy2 - jy1)
        w = jnp.maximum(jnp.minimum(cx2, jx2) - jnp.maximum(cx1, jx1), 0.0)
        h = jnp.maximum(jnp.minimum(cy2, jy2) - jnp.maximum(cy1, jy1), 0.0)
        inter = w * h
        iou = inter / (careas + jareas - inter + 1e-9)
        kr = keep_ref[0:1, pl.ds(jbase, B)]
        return acc + jnp.where(iou > THRESHOLD, 1.0, 0.0) * kr

    acc = jax.lax.fori_loop(0, k, pbody, jnp.zeros((B, B), jnp.float32))
    supp = jnp.sum(acc, axis=1, keepdims=True)
    keep_cur = jnp.reshape((supp == 0.0).astype(jnp.float32), (1, B))

    rx1 = bT_ref[0:1, pl.ds(base, B)]
    ry1 = bT_ref[1:2, pl.ds(base, B)]
    rx2 = bT_ref[2:3, pl.ds(base, B)]
    ry2 = bT_ref[3:4, pl.ds(base, B)]
    rareas = (rx2 - rx1) * (ry2 - ry1)
    bw = jnp.maximum(jnp.minimum(cx2, rx2) - jnp.maximum(cx1, rx1), 0.0)
    bh = jnp.maximum(jnp.minimum(cy2, ry2) - jnp.maximum(cy1, ry1), 0.0)
    binter = bw * bh
    biou = binter / (careas + rareas - binter + 1e-9)
    rowi = jax.lax.broadcasted_iota(jnp.int32, (B, B), 0)
    coli = jax.lax.broadcasted_iota(jnp.int32, (B, B), 1)
    ts = jnp.where((biou > THRESHOLD) & (rowi < coli), 1.0, 0.0)
    l0 = jnp.zeros_like(keep_cur)

    def fcond(carry):
        it, s = carry
        return jnp.logical_and(
            it < B,
            jnp.sum((s[0:1, :] != s[1:2, :]).astype(jnp.float32)) > 0.0)

    def fbody(carry):
        it, s = carry
        r = jnp.dot(s, ts, preferred_element_type=jnp.float32)
        lnew = keep_cur * (r[1:2, :] == 0.0).astype(jnp.float32)
        unew = keep_cur * (r[0:1, :] == 0.0).astype(jnp.float32)
        return it + 1, jnp.concatenate([lnew, unew], axis=0)

    _, s = jax.lax.while_loop(
        fcond, fbody, (0, jnp.concatenate([l0, keep_cur], axis=0)))
    keep_cur = s[0:1, :]

    keep_ref[0:1, pl.ds(base, B)] = keep_cur
    out_ref[0:1, pl.ds(base, B)] = keep_cur


@functools.partial(jax.jit, static_argnames=())
def _nms_keep_mask(bT):
    return pl.pallas_call(
        _nms_step,
        grid=(NB,),
        in_specs=[pl.BlockSpec((4, NPAD), lambda k: (0, 0))],
        out_specs=pl.BlockSpec((1, NPAD), lambda k: (0, 0)),
        out_shape=jax.ShapeDtypeStruct((1, NPAD), jnp.float32),
        scratch_shapes=[pltpu.VMEM((1, NPAD), jnp.float32)],
    )(bT)


def kernel(rois, scores):
    order = jnp.argsort(-scores)
    b = rois[order]
    pad = jnp.full((NPAD - N, 4), -1e8, dtype=jnp.float32)
    bT = jnp.concatenate([b, pad], axis=0).T
    keep = _nms_keep_mask(bT)[0, :N] > 0.5
    kept_sorted_pos = jnp.nonzero(keep, size=MAX_SIZE)[0]
    return order[kept_sorted_pos]


# speedup vs baseline: 1.2633x; 1.2633x over previous
import functools

import jax
import jax.numpy as jnp
from jax.experimental import pallas as pl
from jax.experimental.pallas import tpu as pltpu

N = 5000
NPAD = 5120
B = 256
NB = NPAD // B
THRESHOLD = 0.5
MAX_SIZE = 300


OUTW = 384


def _nms_step(bT_ref, order_ref, out_ref, keep_ref, acc_ref, run_ref):
    k = pl.program_id(0)

    @pl.when(k == 0)
    def _init():
        keep_ref[...] = jnp.zeros_like(keep_ref)
        acc_ref[...] = jnp.zeros_like(acc_ref)
        run_ref[0, 0] = 0.0

    base = k * B
    cx1 = jnp.reshape(bT_ref[0:1, pl.ds(base, B)], (B, 1))
    cy1 = jnp.reshape(bT_ref[1:2, pl.ds(base, B)], (B, 1))
    cx2 = jnp.reshape(bT_ref[2:3, pl.ds(base, B)], (B, 1))
    cy2 = jnp.reshape(bT_ref[3:4, pl.ds(base, B)], (B, 1))
    careas = (cx2 - cx1) * (cy2 - cy1)

    def pbody(jb, acc):
        jbase = jb * B
        jx1 = bT_ref[0:1, pl.ds(jbase, B)]
        jy1 = bT_ref[1:2, pl.ds(jbase, B)]
        jx2 = bT_ref[2:3, pl.ds(jbase, B)]
        jy2 = bT_ref[3:4, pl.ds(jbase, B)]
        jareas = (jx2 - jx1) * (jy2 - jy1)
        w = jnp.maximum(jnp.minimum(cx2, jx2) - jnp.maximum(cx1, jx1), 0.0)
        h = jnp.maximum(jnp.minimum(cy2, jy2) - jnp.maximum(cy1, jy1), 0.0)
        inter = w * h
        iou = inter / (careas + jareas - inter + 1e-9)
        kr = keep_ref[0:1, pl.ds(jbase, B)]
        return acc + jnp.where(iou > THRESHOLD, 1.0, 0.0) * kr

    acc = jax.lax.fori_loop(0, k, pbody, jnp.zeros((B, B), jnp.float32))
    supp = jnp.sum(acc, axis=1, keepdims=True)
    keep_cur = jnp.reshape((supp == 0.0).astype(jnp.float32), (1, B))

    rx1 = bT_ref[0:1, pl.ds(base, B)]
    ry1 = bT_ref[1:2, pl.ds(base, B)]
    rx2 = bT_ref[2:3, pl.ds(base, B)]
    ry2 = bT_ref[3:4, pl.ds(base, B)]
    rareas = (rx2 - rx1) * (ry2 - ry1)
    bw = jnp.maximum(jnp.minimum(cx2, rx2) - jnp.maximum(cx1, rx1), 0.0)
    bh = jnp.maximum(jnp.minimum(cy2, ry2) - jnp.maximum(cy1, ry1), 0.0)
    binter = bw * bh
    biou = binter / (careas + rareas - binter + 1e-9)
    rowi = jax.lax.broadcasted_iota(jnp.int32, (B, B), 0)
    coli = jax.lax.broadcasted_iota(jnp.int32, (B, B), 1)
    ts = jnp.where((biou > THRESHOLD) & (rowi < coli), 1.0, 0.0)
    l0 = jnp.zeros_like(keep_cur)

    def fcond(carry):
        it, s = carry
        return jnp.logical_and(
            it < B,
            jnp.sum((s[0:1, :] != s[1:2, :]).astype(jnp.float32)) > 0.0)

    def fbody(carry):
        it, s = carry
        r = jnp.dot(s, ts, preferred_element_type=jnp.float32)
        lnew = keep_cur * (r[1:2, :] == 0.0).astype(jnp.float32)
        unew = keep_cur * (r[0:1, :] == 0.0).astype(jnp.float32)
        return it + 1, jnp.concatenate([lnew, unew], axis=0)

    _, s = jax.lax.while_loop(
        fcond, fbody, (0, jnp.concatenate([l0, keep_cur], axis=0)))
    keep_cur = s[0:1, :]

    keep_ref[0:1, pl.ds(base, B)] = keep_cur

    lanei = jax.lax.broadcasted_iota(jnp.int32, (1, B), 1)
    validf = jnp.where(base + lanei < N, 1.0, 0.0)
    kv = keep_cur * validf
    ltm = jnp.where(rowi < coli, 1.0, 0.0)
    run = run_ref[0, 0]
    grank = jnp.dot(kv, ltm, preferred_element_type=jnp.float32) + run
    grankt = jnp.reshape(grank, (B, 1))
    kvt = jnp.reshape(kv, (B, 1))
    oiota = jax.lax.broadcasted_iota(jnp.int32, (1, OUTW), 1).astype(
        jnp.float32)
    oneh = jnp.where(grankt == oiota, 1.0, 0.0) * kvt
    ordc = order_ref[0:1, pl.ds(base, B)]
    acc_ref[...] += jnp.dot(ordc, oneh, preferred_element_type=jnp.float32)
    run_ref[0, 0] = run + jnp.sum(kv)

    @pl.when(k == NB - 1)
    def _emit():
        total = run_ref[0, 0]
        o0 = order_ref[0, 0]
        out_ref[...] = acc_ref[...] + jnp.where(oiota >= total, o0, 0.0)


@functools.partial(jax.jit, static_argnames=())
def _nms_kept(bT, orderf):
    return pl.pallas_call(
        _nms_step,
        grid=(NB,),
        in_specs=[
            pl.BlockSpec((4, NPAD), lambda k: (0, 0)),
            pl.BlockSpec((1, NPAD), lambda k: (0, 0)),
        ],
        out_specs=pl.BlockSpec((1, OUTW), lambda k: (0, 0)),
        out_shape=jax.ShapeDtypeStruct((1, OUTW), jnp.float32),
        scratch_shapes=[
            pltpu.VMEM((1, NPAD), jnp.float32),
            pltpu.VMEM((1, OUTW), jnp.float32),
            pltpu.SMEM((1, 1), jnp.float32),
        ],
    )(bT, orderf)


def kernel(rois, scores):
    order = jnp.argsort(-scores)
    b = rois[order]
    pad = jnp.full((NPAD - N, 4), -1e8, dtype=jnp.float32)
    bT = jnp.concatenate([b, pad], axis=0).T
    orderf = jnp.concatenate(
        [order.astype(jnp.float32), jnp.zeros((NPAD - N,), jnp.float32)]
    ).reshape(1, NPAD)
    out = _nms_kept(bT, orderf)
    return out[0, :MAX_SIZE].astype(order.dtype)
